# baseline (device time: 103098 ns/iter reference)
import jax
import jax.numpy as jnp
from jax import lax
from jax.experimental import pallas as pl
from jax.experimental.pallas import tpu as pltpu

N_DEV = 16
S = 2
N_MSG = S * (N_DEV // 2 - 1) + S // 2

RING = (0, 4, 8, 12, 15, 11, 7, 3, 2, 6, 10, 14, 13, 9, 5, 1)
RINGPOS = (0, 15, 8, 7, 1, 14, 9, 6, 2, 13, 10, 5, 3, 12, 11, 4)


def kernel(x):
    m_per, n = x.shape
    m_sub = m_per // S

    my_pos = lax.axis_index("i")
    ring = jnp.array(RING, dtype=jnp.int32)
    r = jnp.array(RINGPOS, dtype=jnp.int32)[my_pos]
    dist = jnp.arange(N_DEV // 2 + 1, dtype=jnp.int32)
    origins_fwd = ring[jnp.mod(r - dist, N_DEV)]
    origins_bwd = ring[jnp.mod(r + dist, N_DEV)]
    nxt = ring[jnp.mod(r + 1, N_DEV)]
    prv = ring[jnp.mod(r - 1, N_DEV)]
    meta = jnp.concatenate(
        [nxt[None], prv[None], origins_fwd, origins_bwd]
    ).astype(jnp.int32)
    n_dist = N_DEV // 2 + 1

    def body(x_ref, meta_ref, out_ref, send_r, recv_r, send_l, recv_l):
        me = lax.axis_index("i")
        nxt = meta_ref[0]
        prv = meta_ref[1]

        def o_fwd(j):
            return meta_ref[2 + j]

        def o_bwd(j):
            return meta_ref[2 + n_dist + j]

        barrier_sem = pltpu.get_barrier_semaphore()
        pl.semaphore_signal(
            barrier_sem, inc=1, device_id=(nxt,),
            device_id_type=pl.DeviceIdType.MESH,
        )
        pl.semaphore_signal(
            barrier_sem, inc=1, device_id=(prv,),
            device_id_type=pl.DeviceIdType.MESH,
        )
        pl.semaphore_wait(barrier_sem, 2)

        def out_sub(origin, s):
            return out_ref.at[pl.ds(origin * m_per + s * m_sub, m_sub), :]

        out_r, in_r, out_l, in_l = [], [], [], []
        for k in range(N_MSG):
            j = k // S
            sr = k % S
            sl = S - 1 - sr
            src_r = (x_ref.at[pl.ds(sr * m_sub, m_sub), :] if j == 0
                     else out_sub(o_fwd(j), sr))
            src_l = (x_ref.at[pl.ds(sl * m_sub, m_sub), :] if j == 0
                     else out_sub(o_bwd(j), sl))
            out_r.append(pltpu.make_async_remote_copy(
                src_ref=src_r, dst_ref=out_sub(o_fwd(j), sr),
                send_sem=send_r.at[k], recv_sem=recv_r.at[k],
                device_id=(nxt,), device_id_type=pl.DeviceIdType.MESH,
            ))
            in_r.append(pltpu.make_async_remote_copy(
                src_ref=out_sub(o_fwd(j + 1), sr),
                dst_ref=out_sub(o_fwd(j + 1), sr),
                send_sem=send_r.at[k], recv_sem=recv_r.at[k],
                device_id=(nxt,), device_id_type=pl.DeviceIdType.MESH,
            ))
            out_l.append(pltpu.make_async_remote_copy(
                src_ref=src_l, dst_ref=out_sub(o_bwd(j), sl),
                send_sem=send_l.at[k], recv_sem=recv_l.at[k],
                device_id=(prv,), device_id_type=pl.DeviceIdType.MESH,
            ))
            in_l.append(pltpu.make_async_remote_copy(
                src_ref=out_sub(o_bwd(j + 1), sl),
                dst_ref=out_sub(o_bwd(j + 1), sl),
                send_sem=send_l.at[k], recv_sem=recv_l.at[k],
                device_id=(prv,), device_id_type=pl.DeviceIdType.MESH,
            ))

        for k in range(S):
            out_r[k].start()
            out_l[k].start()
        out_ref[pl.ds(me * m_per, m_per), :] = x_ref[:, :]

        for k in range(S, N_MSG):
            in_r[k - S].wait_recv()
            out_r[k].start()
            in_l[k - S].wait_recv()
            out_l[k].start()

        for k in range(N_MSG - S, N_MSG):
            in_r[k].wait_recv()
            in_l[k].wait_recv()
        for k in range(N_MSG):
            out_r[k].wait_send()
            out_l[k].wait_send()

    return pl.pallas_call(
        body,
        out_shape=jax.ShapeDtypeStruct((N_DEV * m_per, n), x.dtype),
        in_specs=[
            pl.BlockSpec(memory_space=pltpu.VMEM),
            pl.BlockSpec(memory_space=pltpu.SMEM),
        ],
        out_specs=pl.BlockSpec(memory_space=pltpu.VMEM),
        scratch_shapes=[
            pltpu.SemaphoreType.DMA((N_MSG,)),
            pltpu.SemaphoreType.DMA((N_MSG,)),
            pltpu.SemaphoreType.DMA((N_MSG,)),
            pltpu.SemaphoreType.DMA((N_MSG,)),
        ],
        compiler_params=pltpu.CompilerParams(collective_id=0),
    )(x, meta)


# device time: 96440 ns/iter; 1.0690x vs baseline; 1.0690x over previous
import jax
import jax.numpy as jnp
from jax import lax
from jax.experimental import pallas as pl
from jax.experimental.pallas import tpu as pltpu

N_DEV = 16
S = 2
N_MSG = S * (N_DEV // 2 - 1) + S // 2


def kernel(x):
    m_per, n = x.shape
    m_sub = m_per // S

    def body(x_ref, out_ref, send_r, recv_r, send_l, recv_l):
        me = lax.axis_index("i")

        def ring_id(p):
            q = lax.rem(p + 2 * N_DEV, N_DEV)
            quad = q // 4
            zq = lax.rem(q, 4)
            xc = jnp.where(quad >= 2, 1, 0)
            yc = jnp.where((quad == 1) | (quad == 2), 1, 0)
            zc = jnp.where((quad == 1) | (quad == 3), 3 - zq, zq)
            w = 2 * yc + (xc ^ yc)
            return 4 * zc + w

        zme = me // 4
        wme = lax.rem(me, 4)
        xme = jnp.where((wme == 1) | (wme == 2), 1, 0)
        yme = jnp.where((wme == 2) | (wme == 3), 1, 0)
        r = jnp.where(
            (xme == 0) & (yme == 0), zme,
            jnp.where(
                (xme == 0) & (yme == 1), 7 - zme,
                jnp.where((xme == 1) & (yme == 1), 8 + zme, 15 - zme),
            ),
        )
        nxt = ring_id(r + 1)
        prv = ring_id(r - 1)

        barrier_sem = pltpu.get_barrier_semaphore()
        pl.semaphore_signal(
            barrier_sem, inc=1, device_id=(nxt,),
            device_id_type=pl.DeviceIdType.MESH,
        )
        pl.semaphore_signal(
            barrier_sem, inc=1, device_id=(prv,),
            device_id_type=pl.DeviceIdType.MESH,
        )
        pl.semaphore_wait(barrier_sem, 2)

        def out_sub(origin, s):
            return out_ref.at[pl.ds(origin * m_per + s * m_sub, m_sub), :]

        out_r, in_r, out_l, in_l = [], [], [], []
        for k in range(N_MSG):
            j = k // S
            sr = k % S
            sl = S - 1 - sr
            o_fwd = ring_id(r - j)
            o_fwd_in = ring_id(r - j - 1)
            o_bwd = ring_id(r + j)
            o_bwd_in = ring_id(r + j + 1)
            src_r = (x_ref.at[pl.ds(sr * m_sub, m_sub), :] if j == 0
                     else out_sub(o_fwd, sr))
            src_l = (x_ref.at[pl.ds(sl * m_sub, m_sub), :] if j == 0
                     else out_sub(o_bwd, sl))
            out_r.append(pltpu.make_async_remote_copy(
                src_ref=src_r, dst_ref=out_sub(o_fwd, sr),
                send_sem=send_r.at[k], recv_sem=recv_r.at[k],
                device_id=(nxt,), device_id_type=pl.DeviceIdType.MESH,
            ))
            in_r.append(pltpu.make_async_remote_copy(
                src_ref=out_sub(o_fwd_in, sr),
                dst_ref=out_sub(o_fwd_in, sr),
                send_sem=send_r.at[k], recv_sem=recv_r.at[k],
                device_id=(nxt,), device_id_type=pl.DeviceIdType.MESH,
            ))
            out_l.append(pltpu.make_async_remote_copy(
                src_ref=src_l, dst_ref=out_sub(o_bwd, sl),
                send_sem=send_l.at[k], recv_sem=recv_l.at[k],
                device_id=(prv,), device_id_type=pl.DeviceIdType.MESH,
            ))
            in_l.append(pltpu.make_async_remote_copy(
                src_ref=out_sub(o_bwd_in, sl),
                dst_ref=out_sub(o_bwd_in, sl),
                send_sem=send_l.at[k], recv_sem=recv_l.at[k],
                device_id=(prv,), device_id_type=pl.DeviceIdType.MESH,
            ))

        for k in range(S):
            out_r[k].start()
            out_l[k].start()
        out_ref[pl.ds(me * m_per, m_per), :] = x_ref[:, :]

        for k in range(S, N_MSG):
            in_r[k - S].wait_recv()
            out_r[k].start()
            in_l[k - S].wait_recv()
            out_l[k].start()

        for k in range(N_MSG - S, N_MSG):
            in_r[k].wait_recv()
            in_l[k].wait_recv()
        for k in range(N_MSG):
            out_r[k].wait_send()
            out_l[k].wait_send()

    return pl.pallas_call(
        body,
        out_shape=jax.ShapeDtypeStruct((N_DEV * m_per, n), x.dtype),
        in_specs=[pl.BlockSpec(memory_space=pltpu.VMEM)],
        out_specs=pl.BlockSpec(memory_space=pltpu.VMEM),
        scratch_shapes=[
            pltpu.SemaphoreType.DMA((N_MSG,)),
            pltpu.SemaphoreType.DMA((N_MSG,)),
            pltpu.SemaphoreType.DMA((N_MSG,)),
            pltpu.SemaphoreType.DMA((N_MSG,)),
        ],
        compiler_params=pltpu.CompilerParams(collective_id=0),
    )(x)
